# baseline (speedup 1.0000x reference)
import jax
import jax.numpy as jnp
from jax import lax
from jax.experimental import pallas as pl
from jax.experimental.pallas import tpu as pltpu
from jax.experimental.pallas import tpu_sc as plsc

_VOCAB = 1000000
_EMBED = 64
_B = 16384
_L = 50

_NW = 32
_NTC_FULL = _VOCAB // 128
_TPW = _NTC_FULL // _NW
_TAIL_V0 = _NTC_FULL * 128

_CPARAMS = pltpu.CompilerParams(
    use_tc_tiling_on_sc=True, needs_layout_passes=False,
    disable_bounds_checks=True, disable_semaphore_checks=True)
_MESH = dict(core_axis_name="c", subcore_axis_name="s")


def _wid():
  return lax.axis_index("s") * 2 + lax.axis_index("c")


def _body_a(tab_t, scratch, slab_v, slab64_v, stage_v, gs0, gs1, os0, os1):
  gsem = (gs0, gs1)
  osem = (os0, os1)
  wid = _wid()
  base = wid * _TPW
  ii = lax.iota(jnp.int32, 16)
  i8d = ii // 8
  i8m = ii % 8

  def fire_slab(tc, s):
    for rb in range(8):
      pltpu.async_copy(
          tab_t.at[pl.ds(8 * rb, 8), pl.ds(tc * 128, 128)],
          slab_v.at[s, rb], gsem[s])

  def drain_slab(tc, s):
    for rb in range(8):
      pltpu.make_async_copy(
          tab_t.at[pl.ds(8 * rb, 8), pl.ds(tc * 128, 128)],
          slab_v.at[s, rb], gsem[s]).wait()

  def transpose(s):
    rcs = [2 * c + i8d for c in range(4)]

    @pl.loop(0, 128, step=4)
    def _(vc0):
      vals = []
      for dv in range(4):
        vcs = jnp.full((16,), 0, jnp.int32) + (vc0 + dv)
        for c in range(4):
          vals.append(plsc.load_gather(slab_v.at[s], [rcs[c], i8m, vcs]))
      for dv in range(4):
        for c in range(4):
          stage_v[s, vc0 + dv, pl.ds(16 * c, 16)] = vals[4 * dv + c]

  def fire_write(tc, s):
    pltpu.async_copy(stage_v.at[s], scratch.at[pl.ds(tc * 128, 128)], osem[s])

  def drain_write(tc, s):
    pltpu.make_async_copy(
        stage_v.at[s], scratch.at[pl.ds(tc * 128, 128)], osem[s]).wait()

  def unit(i, s, drainw):
    tc = base + i
    if drainw:
      drain_write(tc - 2, s)
    fire_slab(tc, s)
    drain_slab(tc - 1, 1 - s)
    transpose(1 - s)
    fire_write(tc - 1, 1 - s)

  fire_slab(base, 0)
  unit(1, 1, drainw=False)
  unit(2, 0, drainw=True)

  @pl.loop(3, _TPW - 3, step=2)
  def _(ib):
    for d in range(2):
      unit(ib + d, (1 + d) % 2, drainw=True)

  unit(_TPW - 3, 1, drainw=True)
  unit(_TPW - 2, 0, drainw=True)
  unit(_TPW - 1, 1, drainw=True)
  drain_slab(base + _TPW - 1, 1)
  transpose(1)
  fire_write(base + _TPW - 1, 1)
  drain_write(base + _TPW - 2, 0)
  drain_write(base + _TPW - 1, 1)

  @pl.when(wid < 4)
  def _():
    tc = _NTC_FULL - 4 + wid
    fire_slab(tc, 0)
    drain_slab(tc, 0)
    transpose(0)
    pltpu.sync_copy(stage_v.at[0], scratch.at[pl.ds(tc * 128, 128)])

  @pl.when(wid == 4)
  def _():
    for rb in range(8):
      pltpu.async_copy(
          tab_t.at[pl.ds(8 * rb, 8), pl.ds(_TAIL_V0, 64)],
          slab64_v.at[rb], gs0)
    for rb in range(8):
      pltpu.make_async_copy(
          tab_t.at[pl.ds(8 * rb, 8), pl.ds(_TAIL_V0, 64)],
          slab64_v.at[rb], gs0).wait()

    @pl.loop(0, 64, step=4)
    def _(vc0):
      vals = []
      for dv in range(4):
        vcs = jnp.full((16,), 0, jnp.int32) + (vc0 + dv)
        for c in range(4):
          vals.append(plsc.load_gather(slab64_v, [2 * c + i8d, i8m, vcs]))
      for dv in range(4):
        for c in range(4):
          stage_v[0, vc0 + dv, pl.ds(16 * c, 16)] = vals[4 * dv + c]

    pltpu.sync_copy(stage_v.at[0, pl.ds(0, 64)],
                    scratch.at[pl.ds(_TAIL_V0, 64)])


def _body_b(tok3, scratch, out_t, idx_v, rows_v, ostage_v,
            is0, is1, gs0, gs1, os0, os1):
  isem = (is0, is1)
  gsem = (gs0, gs1)
  osem = (os0, os1)
  wid = _wid()
  q = wid // 2
  off = (wid % 2) * 4
  bb0 = wid * 4
  bi = lax.iota(jnp.int32, 16)

  def fire_iload(l, sl):
    pltpu.async_copy(tok3.at[l, pl.ds(8 * q, 8)], idx_v.at[sl], isem[sl])

  def drain_iload(l, sl):
    pltpu.make_async_copy(
        tok3.at[l, pl.ds(8 * q, 8)], idx_v.at[sl], isem[sl]).wait()

  def fire_g(j, s, sl):
    pltpu.async_copy(scratch.at[idx_v.at[sl, off + j]], rows_v.at[s], gsem[s])

  def drain_g(j, s, sl):
    pltpu.make_async_copy(
        scratch.at[idx_v.at[sl, off + j]], rows_v.at[s], gsem[s]).wait()

  bks = [16 * k + bi for k in range(8)]

  def transpose(s):
    @pl.loop(0, 64, step=2)
    def _(e0):
      vals = []
      for de in range(2):
        es = jnp.full((16,), 0, jnp.int32) + (e0 + de)
        for k in range(8):
          vals.append(plsc.load_gather(rows_v.at[s], [bks[k], es]))
      for de in range(2):
        for k in range(8):
          ostage_v[s, e0 + de, pl.ds(16 * k, 16)] = vals[8 * de + k]

  def out_ref(l, j):
    return out_t.at[l, :, pl.ds((bb0 + j) * 128, 128)]

  def fire_w(l, j, s):
    pltpu.async_copy(ostage_v.at[s], out_ref(l, j), osem[s])

  def drain_w(l, j, s):
    pltpu.make_async_copy(ostage_v.at[s], out_ref(l, j), osem[s]).wait()

  def unit(l, j, sl, prev, drainw=True, iload_next=True):
    s = j % 2
    if j == 0:
      drain_iload(l, sl)
    if drainw:
      if j >= 2:
        drain_w(l, j - 2, s)
      else:
        drain_w(l - 1, j + 2, s)
    fire_g(j, s, sl)
    lp, jp, slp = prev
    drain_g(jp, 1 - s, slp)
    if j == 0 and iload_next:
      fire_iload(l + 1, 1 - sl)
    transpose(1 - s)
    fire_w(lp, jp, 1 - s)

  pltpu.sync_copy(tok3.at[0, pl.ds(8 * q, 8)], idx_v.at[0])
  fire_g(0, 0, 0)
  fire_iload(1, 1)
  unit(0, 1, 0, (0, 0, 0), drainw=False)
  unit(0, 2, 0, (0, 1, 0))
  unit(0, 3, 0, (0, 2, 0))

  @pl.loop(1, _L - 1, step=2)
  def _(lb):
    for d in range(2):
      l = lb + d
      sl = (1 + d) % 2
      unit(l, 0, sl, (l - 1, 3, 1 - sl))
      unit(l, 1, sl, (l, 0, sl))
      unit(l, 2, sl, (l, 1, sl))
      unit(l, 3, sl, (l, 2, sl))

  unit(_L - 1, 0, 1, (_L - 2, 3, 0), iload_next=False)
  unit(_L - 1, 1, 1, (_L - 1, 0, 1))
  unit(_L - 1, 2, 1, (_L - 1, 1, 1))
  unit(_L - 1, 3, 1, (_L - 1, 2, 1))
  drain_g(3, 1, 1)
  transpose(1)
  fire_w(_L - 1, 3, 1)
  drain_w(_L - 1, 2, 0)
  drain_w(_L - 1, 3, 1)


@jax.jit
def _embed(tok3, tab_t):
  scratch = pl.kernel(
      _body_a,
      out_type=jax.ShapeDtypeStruct((_VOCAB, 128), jnp.float32),
      mesh=plsc.VectorSubcoreMesh(**_MESH),
      compiler_params=_CPARAMS,
      scratch_types=[
          pltpu.VMEM((2, 8, 8, 128), jnp.float32),
          pltpu.VMEM((8, 8, 64), jnp.float32),
          pltpu.VMEM((2, 128, 128), jnp.float32),
          pltpu.SemaphoreType.DMA,
          pltpu.SemaphoreType.DMA,
          pltpu.SemaphoreType.DMA,
          pltpu.SemaphoreType.DMA,
      ],
  )(tab_t)
  out_t = pl.kernel(
      _body_b,
      out_type=jax.ShapeDtypeStruct((_L, _EMBED, _B), jnp.float32),
      mesh=plsc.VectorSubcoreMesh(**_MESH),
      compiler_params=_CPARAMS,
      scratch_types=[
          pltpu.VMEM((2, 8, 128), jnp.int32),
          pltpu.VMEM((2, 128, 128), jnp.float32),
          pltpu.VMEM((2, _EMBED, 128), jnp.float32),
          pltpu.SemaphoreType.DMA,
          pltpu.SemaphoreType.DMA,
          pltpu.SemaphoreType.DMA,
          pltpu.SemaphoreType.DMA,
          pltpu.SemaphoreType.DMA,
          pltpu.SemaphoreType.DMA,
      ],
  )(tok3, scratch)
  return out_t


def kernel(tokens, table):
  tok3 = tokens.T.astype(jnp.int32).reshape(_L, _B // 128, 128)
  out_t = _embed(tok3, table.T)
  return out_t.transpose(2, 0, 1)


# speedup vs baseline: 1.0313x; 1.0313x over previous
import jax
import jax.numpy as jnp
from jax import lax
from jax.experimental import pallas as pl
from jax.experimental.pallas import tpu as pltpu
from jax.experimental.pallas import tpu_sc as plsc

_VOCAB = 1000000
_EMBED = 64
_B = 16384
_L = 50

_NW = 32
_NTC_FULL = _VOCAB // 128
_NU = _NTC_FULL // 2
_UPW = _NU // _NW
_TAIL_V0 = _NTC_FULL * 128

_CPARAMS = pltpu.CompilerParams(
    use_tc_tiling_on_sc=True, needs_layout_passes=False,
    disable_bounds_checks=True, disable_semaphore_checks=True)
_MESH = dict(core_axis_name="c", subcore_axis_name="s")


def _wid():
  return lax.axis_index("s") * 2 + lax.axis_index("c")


def _body_a(tab_t, scratch, slab_v, slab64_v, stage_v, gs0, gs1, os0, os1):
  gsem = (gs0, gs1)
  osem = (os0, os1)
  wid = _wid()
  base = wid * _UPW
  ii = lax.iota(jnp.int32, 16)
  i8d = ii // 8
  i8m = ii % 8
  rcs = [16 * c + ii for c in range(4)]

  def fire_slab(u, s):
    pltpu.async_copy(tab_t.at[:, pl.ds(u * 256, 256)], slab_v.at[s], gsem[s])

  def drain_slab(u, s):
    pltpu.make_async_copy(
        tab_t.at[:, pl.ds(u * 256, 256)], slab_v.at[s], gsem[s]).wait()

  def transpose(s):
    @pl.loop(0, 256, step=4)
    def _(vc0):
      vals = []
      for dv in range(4):
        vcs = jnp.full((16,), 0, jnp.int32) + (vc0 + dv)
        for c in range(4):
          vals.append(plsc.load_gather(slab_v.at[s], [rcs[c], vcs]))
      for dv in range(4):
        for c in range(4):
          stage_v[s, vc0 + dv, pl.ds(16 * c, 16)] = vals[4 * dv + c]

  def fire_write(u, s):
    pltpu.async_copy(stage_v.at[s], scratch.at[pl.ds(u * 256, 256)], osem[s])

  def drain_write(u, s):
    pltpu.make_async_copy(
        stage_v.at[s], scratch.at[pl.ds(u * 256, 256)], osem[s]).wait()

  def unit(i, s, drainw):
    u = base + i
    if drainw:
      drain_write(u - 2, s)
    fire_slab(u, s)
    drain_slab(u - 1, 1 - s)
    transpose(1 - s)
    fire_write(u - 1, 1 - s)

  fire_slab(base, 0)
  unit(1, 1, drainw=False)
  unit(2, 0, drainw=True)

  @pl.loop(3, _UPW - 3, step=2)
  def _(ib):
    for d in range(2):
      unit(ib + d, (1 + d) % 2, drainw=True)

  unit(_UPW - 3, 1, drainw=True)
  unit(_UPW - 2, 0, drainw=True)
  unit(_UPW - 1, 1, drainw=True)
  drain_slab(base + _UPW - 1, 1)
  transpose(1)
  fire_write(base + _UPW - 1, 1)
  drain_write(base + _UPW - 2, 0)
  drain_write(base + _UPW - 1, 1)

  @pl.when(wid < 2)
  def _():
    u = _NU - 2 + wid
    fire_slab(u, 0)
    drain_slab(u, 0)
    transpose(0)
    pltpu.sync_copy(stage_v.at[0], scratch.at[pl.ds(u * 256, 256)])

  @pl.when(wid == 4)
  def _():
    for rb in range(8):
      pltpu.async_copy(
          tab_t.at[pl.ds(8 * rb, 8), pl.ds(_TAIL_V0, 64)],
          slab64_v.at[rb], gs0)
    for rb in range(8):
      pltpu.make_async_copy(
          tab_t.at[pl.ds(8 * rb, 8), pl.ds(_TAIL_V0, 64)],
          slab64_v.at[rb], gs0).wait()

    @pl.loop(0, 64, step=4)
    def _(vc0):
      vals = []
      for dv in range(4):
        vcs = jnp.full((16,), 0, jnp.int32) + (vc0 + dv)
        for c in range(4):
          vals.append(plsc.load_gather(slab64_v, [2 * c + i8d, i8m, vcs]))
      for dv in range(4):
        for c in range(4):
          stage_v[0, vc0 + dv, pl.ds(16 * c, 16)] = vals[4 * dv + c]

    pltpu.sync_copy(stage_v.at[0, pl.ds(0, 64)],
                    scratch.at[pl.ds(_TAIL_V0, 64)])


def _body_b(tok3, scratch, out_t, idx_v, rows_v, ostage_v,
            is0, is1, gs0, gs1, gs2, gs3, os0, os1):
  isem = (is0, is1)
  gsem = (gs0, gs1, gs2, gs3)
  osem = (os0, os1)
  wid = _wid()
  q = wid // 2
  off = (wid % 2) * 4
  bb0 = wid * 4
  bi = lax.iota(jnp.int32, 16)
  bks = [16 * k + bi for k in range(8)]

  def fire_iload(l, sl):
    pltpu.async_copy(tok3.at[l, pl.ds(8 * q, 8)], idx_v.at[sl], isem[sl])

  def drain_iload(l, sl):
    pltpu.make_async_copy(
        tok3.at[l, pl.ds(8 * q, 8)], idx_v.at[sl], isem[sl]).wait()

  def fire_g(j, sl):
    pltpu.async_copy(scratch.at[idx_v.at[sl, off + j]], rows_v.at[j], gsem[j])

  def drain_g(j, sl):
    pltpu.make_async_copy(
        scratch.at[idx_v.at[sl, off + j]], rows_v.at[j], gsem[j]).wait()

  def transpose(j, o):
    @pl.loop(0, 64, step=2)
    def _(e0):
      vals = []
      for de in range(2):
        es = jnp.full((16,), 0, jnp.int32) + (e0 + de)
        for k in range(8):
          vals.append(plsc.load_gather(rows_v.at[j], [bks[k], es]))
      for de in range(2):
        for k in range(8):
          ostage_v[o, e0 + de, pl.ds(16 * k, 16)] = vals[8 * de + k]

  def out_ref(l, j):
    return out_t.at[l, :, pl.ds((bb0 + j) * 128, 128)]

  def fire_w(l, j, o):
    pltpu.async_copy(ostage_v.at[o], out_ref(l, j), osem[o])

  def drain_w(l, j, o):
    pltpu.make_async_copy(ostage_v.at[o], out_ref(l, j), osem[o]).wait()

  def pos(l, sl, nsl, first=False, fire_next=True, iload2=True):
    for j in range(4):
      o = j % 2
      drain_g(j, sl)
      if not (first and j < 2):
        if j >= 2:
          drain_w(l, j - 2, o)
        else:
          drain_w(l - 1, j + 2, o)
      transpose(j, o)
      if fire_next:
        if j == 0:
          drain_iload(l + 1, nsl)
        fire_g(j, nsl)
      if j == 3 and iload2:
        fire_iload(l + 2, sl)
      fire_w(l, j, o)

  pltpu.sync_copy(tok3.at[0, pl.ds(8 * q, 8)], idx_v.at[0])
  for j in range(4):
    fire_g(j, 0)
  fire_iload(1, 1)
  pos(0, 0, 1, first=True)

  @pl.loop(1, _L - 3, step=2)
  def _(lb):
    pos(lb, 1, 0)
    pos(lb + 1, 0, 1)

  pos(_L - 3, 1, 0, iload2=True)
  pos(_L - 2, 0, 1, iload2=False)
  pos(_L - 1, 1, 0, fire_next=False, iload2=False)
  drain_w(_L - 1, 2, 0)
  drain_w(_L - 1, 3, 1)


@jax.jit
def _embed(tok3, tab_t):
  scratch = pl.kernel(
      _body_a,
      out_type=jax.ShapeDtypeStruct((_VOCAB, 128), jnp.float32),
      mesh=plsc.VectorSubcoreMesh(**_MESH),
      compiler_params=_CPARAMS,
      scratch_types=[
          pltpu.VMEM((2, _EMBED, 256), jnp.float32),
          pltpu.VMEM((8, 8, 64), jnp.float32),
          pltpu.VMEM((2, 256, 128), jnp.float32),
          pltpu.SemaphoreType.DMA,
          pltpu.SemaphoreType.DMA,
          pltpu.SemaphoreType.DMA,
          pltpu.SemaphoreType.DMA,
      ],
  )(tab_t)
  out_t = pl.kernel(
      _body_b,
      out_type=jax.ShapeDtypeStruct((_L, _EMBED, _B), jnp.float32),
      mesh=plsc.VectorSubcoreMesh(**_MESH),
      compiler_params=_CPARAMS,
      scratch_types=[
          pltpu.VMEM((2, 8, 128), jnp.int32),
          pltpu.VMEM((4, 128, 128), jnp.float32),
          pltpu.VMEM((2, _EMBED, 128), jnp.float32),
          pltpu.SemaphoreType.DMA,
          pltpu.SemaphoreType.DMA,
          pltpu.SemaphoreType.DMA,
          pltpu.SemaphoreType.DMA,
          pltpu.SemaphoreType.DMA,
          pltpu.SemaphoreType.DMA,
          pltpu.SemaphoreType.DMA,
          pltpu.SemaphoreType.DMA,
      ],
  )(tok3, scratch)
  return out_t


def kernel(tokens, table):
  tok3 = tokens.T.astype(jnp.int32).reshape(_L, _B // 128, 128)
  out_t = _embed(tok3, table.T)
  return out_t.transpose(2, 0, 1)


# speedup vs baseline: 3.7033x; 3.5911x over previous
import jax
import jax.numpy as jnp
from jax import lax
from jax.experimental import pallas as pl
from jax.experimental.pallas import tpu as pltpu
from jax.experimental.pallas import tpu_sc as plsc

_VOCAB = 1000000
_EMBED = 64
_B = 16384
_L = 50

_NW = 32
_NTC_FULL = _VOCAB // 128
_NU = _NTC_FULL // 2
_UPW = _NU // _NW
_TAIL_V0 = _NTC_FULL * 128

_CPARAMS = pltpu.CompilerParams(
    use_tc_tiling_on_sc=True, needs_layout_passes=False,
    disable_bounds_checks=True, disable_semaphore_checks=True)
_MESH = dict(core_axis_name="c", subcore_axis_name="s")


def _wid():
  return lax.axis_index("s") * 2 + lax.axis_index("c")


def _body_a(tab_t, scratch, slab_v, slab64_v, stage_v, gs0, gs1, os0, os1):
  gsem = (gs0, gs1)
  osem = (os0, os1)
  wid = _wid()
  base = wid * _UPW
  ii = lax.iota(jnp.int32, 16)
  i8d = ii // 8
  i8m = ii % 8
  rcs = [16 * c + ii for c in range(4)]

  def fire_slab(u, s):
    pltpu.async_copy(tab_t.at[:, pl.ds(u * 256, 256)], slab_v.at[s], gsem[s])

  def drain_slab(u, s):
    pltpu.make_async_copy(
        tab_t.at[:, pl.ds(u * 256, 256)], slab_v.at[s], gsem[s]).wait()

  rot = [(ii + d) % 16 for d in range(16)]

  def transpose(s):
    @pl.loop(0, 256, step=16)
    def _(vc0):
      vcs = vc0 + ii
      for e0 in range(0, _EMBED, 16):
        vals = []
        for d in range(16):
          vals.append(plsc.load_gather(slab_v.at[s], [e0 + rot[d], vcs]))
        for d in range(16):
          plsc.store_scatter(stage_v.at[s], [vcs, e0 + rot[d]], vals[d])

  def fire_write(u, s):
    pltpu.async_copy(stage_v.at[s], scratch.at[pl.ds(u * 256, 256)], osem[s])

  def drain_write(u, s):
    pltpu.make_async_copy(
        stage_v.at[s], scratch.at[pl.ds(u * 256, 256)], osem[s]).wait()

  def unit(i, s, drainw):
    u = base + i
    if drainw:
      drain_write(u - 2, s)
    fire_slab(u, s)
    drain_slab(u - 1, 1 - s)
    transpose(1 - s)
    fire_write(u - 1, 1 - s)

  fire_slab(base, 0)
  unit(1, 1, drainw=False)
  unit(2, 0, drainw=True)

  @pl.loop(3, _UPW - 3, step=2)
  def _(ib):
    for d in range(2):
      unit(ib + d, (1 + d) % 2, drainw=True)

  unit(_UPW - 3, 1, drainw=True)
  unit(_UPW - 2, 0, drainw=True)
  unit(_UPW - 1, 1, drainw=True)
  drain_slab(base + _UPW - 1, 1)
  transpose(1)
  fire_write(base + _UPW - 1, 1)
  drain_write(base + _UPW - 2, 0)
  drain_write(base + _UPW - 1, 1)

  @pl.when(wid < 2)
  def _():
    u = _NU - 2 + wid
    fire_slab(u, 0)
    drain_slab(u, 0)
    transpose(0)
    pltpu.sync_copy(stage_v.at[0], scratch.at[pl.ds(u * 256, 256)])

  @pl.when(wid == 4)
  def _():
    for rb in range(8):
      pltpu.async_copy(
          tab_t.at[pl.ds(8 * rb, 8), pl.ds(_TAIL_V0, 64)],
          slab64_v.at[rb], gs0)
    for rb in range(8):
      pltpu.make_async_copy(
          tab_t.at[pl.ds(8 * rb, 8), pl.ds(_TAIL_V0, 64)],
          slab64_v.at[rb], gs0).wait()

    @pl.loop(0, 64, step=4)
    def _(vc0):
      vals = []
      for dv in range(4):
        vcs = jnp.full((16,), 0, jnp.int32) + (vc0 + dv)
        for c in range(4):
          vals.append(plsc.load_gather(slab64_v, [2 * c + i8d, i8m, vcs]))
      for dv in range(4):
        for c in range(4):
          stage_v[0, vc0 + dv, pl.ds(16 * c, 16)] = vals[4 * dv + c]

    pltpu.sync_copy(stage_v.at[0, pl.ds(0, 64)],
                    scratch.at[pl.ds(_TAIL_V0, 64)])


def _body_b(tok3, scratch, out_t, idx_v, rows_v, ostage_v,
            isem, gs0, gs1, gs2, gs3, os0, os1):
  gsem = (gs0, gs1, gs2, gs3)
  osem = (os0, os1)
  wid = _wid()
  q = wid // 2
  off = (wid % 2) * 4
  bb0 = wid * 4
  bi = lax.iota(jnp.int32, 16)
  bks = [16 * k + bi for k in range(8)]

  def fire_iload(l, sl):
    pltpu.async_copy(tok3.at[l, pl.ds(8 * q, 8)], idx_v.at[sl], isem)

  def drain_iload(l, sl):
    pltpu.make_async_copy(
        tok3.at[l, pl.ds(8 * q, 8)], idx_v.at[sl], isem).wait()

  def fire_g(j, sl):
    pltpu.async_copy(scratch.at[idx_v.at[sl, off + j]], rows_v.at[j], gsem[j])

  def drain_g(j, sl):
    pltpu.make_async_copy(
        scratch.at[idx_v.at[sl, off + j]], rows_v.at[j], gsem[j]).wait()

  rot = [(bi + d) % 16 for d in range(16)]

  def transpose(j, o):
    @pl.loop(0, 128, step=16)
    def _(b0):
      bs = b0 + bi
      for e0 in range(0, _EMBED, 16):
        vals = []
        for d in range(16):
          vals.append(plsc.load_gather(rows_v.at[j], [bs, e0 + rot[d]]))
        for d in range(16):
          plsc.store_scatter(ostage_v.at[o], [e0 + rot[d], bs], vals[d])

  def out_ref(l, j):
    return out_t.at[l, :, pl.ds((bb0 + j) * 128, 128)]

  def fire_w(l, j, o):
    pltpu.async_copy(ostage_v.at[o], out_ref(l, j), osem[o])

  def drain_w(l, j, o):
    pltpu.make_async_copy(ostage_v.at[o], out_ref(l, j), osem[o]).wait()

  def pos(l, sl, nsl, first=False, fire_next=True, iload2=True):
    for j in range(4):
      o = j % 2
      drain_g(j, sl)
      if not (first and j < 2):
        if j >= 2:
          drain_w(l, j - 2, o)
        else:
          drain_w(l - 1, j + 2, o)
      transpose(j, o)
      if fire_next:
        if j == 0:
          drain_iload(l + 1, nsl)
        fire_g(j, nsl)
      if j == 3 and iload2:
        fire_iload(jnp.minimum(l + 2, _L - 1), sl)
      fire_w(l, j, o)

  pltpu.sync_copy(tok3.at[0, pl.ds(8 * q, 8)], idx_v.at[0])
  for j in range(4):
    fire_g(j, 0)
  fire_iload(1, 1)
  pos(0, 0, 1, first=True)

  @pl.loop(1, _L - 1)
  def _(l):
    sl = l % 2
    pos(l, sl, 1 - sl)

  pos(_L - 1, 1, 0, fire_next=False, iload2=False)
  drain_w(_L - 1, 2, 0)
  drain_w(_L - 1, 3, 1)
  drain_iload(_L - 1, 0)


@jax.jit
def _embed(tok3, tab_t):
  scratch = pl.kernel(
      _body_a,
      out_type=jax.ShapeDtypeStruct((_VOCAB, 128), jnp.float32),
      mesh=plsc.VectorSubcoreMesh(**_MESH),
      compiler_params=_CPARAMS,
      scratch_types=[
          pltpu.VMEM((2, _EMBED, 256), jnp.float32),
          pltpu.VMEM((8, 8, 64), jnp.float32),
          pltpu.VMEM((2, 256, 128), jnp.float32),
          pltpu.SemaphoreType.DMA,
          pltpu.SemaphoreType.DMA,
          pltpu.SemaphoreType.DMA,
          pltpu.SemaphoreType.DMA,
      ],
  )(tab_t)
  out_t = pl.kernel(
      _body_b,
      out_type=jax.ShapeDtypeStruct((_L, _EMBED, _B), jnp.float32),
      mesh=plsc.VectorSubcoreMesh(**_MESH),
      compiler_params=_CPARAMS,
      scratch_types=[
          pltpu.VMEM((2, 8, 128), jnp.int32),
          pltpu.VMEM((4, 128, 128), jnp.float32),
          pltpu.VMEM((2, _EMBED, 128), jnp.float32),
          pltpu.SemaphoreType.DMA,
          pltpu.SemaphoreType.DMA,
          pltpu.SemaphoreType.DMA,
          pltpu.SemaphoreType.DMA,
          pltpu.SemaphoreType.DMA,
          pltpu.SemaphoreType.DMA,
          pltpu.SemaphoreType.DMA,
      ],
  )(tok3, scratch)
  return out_t


def kernel(tokens, table):
  tok3 = tokens.T.astype(jnp.int32).reshape(_L, _B // 128, 128)
  out_t = _embed(tok3, table.T)
  return out_t.transpose(2, 0, 1)
